# baseline (speedup 1.0000x reference)
import functools

import jax
import jax.numpy as jnp
from jax import lax
from jax.experimental import pallas as pl
from jax.experimental.pallas import tpu as pltpu
from jax.experimental.pallas import tpu_sc as plsc

B = 16384
D = 768
NCHUNK = 4
BC = B // NCHUNK
_BM = 512


def _sc_gather(stu_ids, exer_in, stu_table, bert_table):
    info = plsc.get_sparse_core_info()
    nw = info.num_cores * info.num_subcores
    nc = info.num_cores
    bpw = BC // nw
    ch = min(64, bpw)
    nch = bpw // ch

    mesh = plsc.VectorSubcoreMesh(core_axis_name="c", subcore_axis_name="s")

    @functools.partial(
        pl.kernel,
        mesh=mesh,
        out_type=[
            jax.ShapeDtypeStruct((BC, D), jnp.float32),
            jax.ShapeDtypeStruct((BC, D), jnp.float32),
        ],
        scratch_types=[
            pltpu.VMEM((bpw,), jnp.int32),
            pltpu.VMEM((bpw,), jnp.int32),
            pltpu.VMEM((ch, D), jnp.float32),
            pltpu.VMEM((ch, D), jnp.float32),
            pltpu.SemaphoreType.DMA,
            pltpu.SemaphoreType.DMA,
            pltpu.SemaphoreType.DMA,
            pltpu.SemaphoreType.DMA,
        ],
    )
    def gather_k(stu_ids_hbm, exer_in_hbm, stu_table_hbm, bert_table_hbm,
                 prof_out, exer_out, idx_s, idx_e, rows_0, rows_1,
                 gsem_0, gsem_1, wsem_0, wsem_1):
        wid = lax.axis_index("s") * nc + lax.axis_index("c")
        base = wid * bpw
        pltpu.sync_copy(stu_ids_hbm.at[pl.ds(base, bpw)], idx_s)
        pltpu.sync_copy(exer_in_hbm.at[pl.ds(base, bpw)], idx_e)

        tasks = []
        for c in range(nch):
            tasks.append((idx_s, stu_table_hbm, prof_out, c * ch))
            tasks.append((idx_e, bert_table_hbm, exer_out, c * ch))

        bufs = (rows_0, rows_1)
        gsems = (gsem_0, gsem_1)
        wsems = (wsem_0, wsem_1)

        def issue_gather(t, slot):
            idx, table, _, off = tasks[t]
            return pltpu.async_copy(
                table.at[idx.at[pl.ds(off, ch)]], bufs[slot], gsems[slot])

        gcp = [None, None]
        wcp = [None, None]
        gcp[0] = issue_gather(0, 0)
        for t in range(len(tasks)):
            slot = t % 2
            nslot = (t + 1) % 2
            if t + 1 < len(tasks):
                if wcp[nslot] is not None:
                    wcp[nslot].wait()
                gcp[nslot] = issue_gather(t + 1, nslot)
            gcp[slot].wait()
            _, _, out, off = tasks[t]
            wcp[slot] = pltpu.async_copy(
                bufs[slot], out.at[pl.ds(base + off, ch)], wsems[slot])
        if wcp[0] is not None:
            wcp[0].wait()
        if wcp[1] is not None:
            wcp[1].wait()

    return gather_k(stu_ids, exer_in, stu_table, bert_table)


def _sigmoid(x):
    return 0.5 * jnp.tanh(0.5 * x) + 0.5


def _mlp_impl(ex_ref, pr_ref, w1a_ref, w1b_ref, b1_ref, w2_ref, b2_ref,
              w3_ref, b3_ref, out_ref, exf_ref, prf_ref):
    ex = ex_ref[...]
    pr = pr_ref[...]
    exf_ref[...] = ex
    prf_ref[...] = pr
    exb = ex.astype(jnp.bfloat16)
    prb = pr.astype(jnp.bfloat16)
    h = (jnp.dot(exb, w1a_ref[...], preferred_element_type=jnp.float32)
         + jnp.dot(prb, w1b_ref[...], preferred_element_type=jnp.float32)
         + b1_ref[...])
    h = _sigmoid(h).astype(jnp.bfloat16)
    h = _sigmoid(
        jnp.dot(h, w2_ref[...], preferred_element_type=jnp.float32)
        + b2_ref[...]).astype(jnp.bfloat16)
    logits = (jnp.dot(h, w3_ref[...], preferred_element_type=jnp.float32)
              + b3_ref[...])
    out_ref[...] = _sigmoid(logits)


def _mlp_body_aliased(ex_ref, pr_ref, w1a_ref, w1b_ref, b1_ref, w2_ref,
                      b2_ref, w3_ref, b3_ref, of_in, ef_in, pf_in,
                      out_ref, exf_ref, prf_ref):
    del of_in, ef_in, pf_in
    _mlp_impl(ex_ref, pr_ref, w1a_ref, w1b_ref, b1_ref, w2_ref, b2_ref,
              w3_ref, b3_ref, out_ref, exf_ref, prf_ref)


_WEIGHT_SPECS = [
    pl.BlockSpec((D, 2 * D), lambda i: (0, 0)),
    pl.BlockSpec((D, 2 * D), lambda i: (0, 0)),
    pl.BlockSpec((1, 2 * D), lambda i: (0, 0)),
    pl.BlockSpec((2 * D, D), lambda i: (0, 0)),
    pl.BlockSpec((1, D), lambda i: (0, 0)),
    pl.BlockSpec((D, 1), lambda i: (0, 0)),
    pl.BlockSpec((1, 1), lambda i: (0, 0)),
]

_OUT_SHAPES = [
    jax.ShapeDtypeStruct((B, 1), jnp.float32),
    jax.ShapeDtypeStruct((B, D), jnp.float32),
    jax.ShapeDtypeStruct((B, D), jnp.float32),
]


def _tc_mlp_chunk(k, ex_k, pr_k, w1a, w1b, b1r, w2, b2r, w3, b3r,
                  out_full=None, exer_full=None, prof_full=None):
    nsteps = BC // _BM
    base = k * nsteps

    data_specs = [
        pl.BlockSpec((_BM, D), lambda i: (i, 0)),
        pl.BlockSpec((_BM, D), lambda i: (i, 0)),
    ]
    out_specs = [
        pl.BlockSpec((_BM, 1), lambda i: (base + i, 0)),
        pl.BlockSpec((_BM, D), lambda i: (base + i, 0)),
        pl.BlockSpec((_BM, D), lambda i: (base + i, 0)),
    ]
    args = (ex_k, pr_k, w1a, w1b, b1r, w2, b2r, w3, b3r)
    if out_full is None:
        return pl.pallas_call(
            _mlp_impl,
            grid=(nsteps,),
            in_specs=data_specs + _WEIGHT_SPECS,
            out_specs=out_specs,
            out_shape=_OUT_SHAPES,
        )(*args)
    _full = pl.BlockSpec(memory_space=pl.ANY)
    return pl.pallas_call(
        _mlp_body_aliased,
        grid=(nsteps,),
        in_specs=data_specs + _WEIGHT_SPECS + [_full, _full, _full],
        out_specs=out_specs,
        out_shape=_OUT_SHAPES,
        input_output_aliases={9: 0, 10: 1, 11: 2},
    )(*args, out_full, exer_full, prof_full)


def kernel(stu_ids, exer_in, bert_table, stu_table, W1, b1, W2, b2, W3, b3):
    w1a = W1[:D].astype(jnp.bfloat16)
    w1b = W1[D:].astype(jnp.bfloat16)
    w2 = W2.astype(jnp.bfloat16)
    w3 = W3.astype(jnp.bfloat16)
    b1r = b1.reshape(1, 2 * D)
    b2r = b2.reshape(1, D)
    b3r = b3.reshape(1, 1)

    chunks = []
    for k in range(NCHUNK):
        sl = slice(k * BC, (k + 1) * BC)
        chunks.append(_sc_gather(stu_ids[sl], exer_in[sl],
                                 stu_table, bert_table))

    out_full = exer_full = prof_full = None
    for k in range(NCHUNK):
        prof_k, exer_k = chunks[k]
        out_full, exer_full, prof_full = _tc_mlp_chunk(
            k, exer_k, prof_k, w1a, w1b, b1r, w2, b2r, w3, b3r,
            out_full=out_full, exer_full=exer_full, prof_full=prof_full)

    return out_full.reshape(B), exer_full, prof_full


# speedup vs baseline: 1.0659x; 1.0659x over previous
import functools

import jax
import jax.numpy as jnp
from jax import lax
from jax.experimental import pallas as pl
from jax.experimental.pallas import tpu as pltpu
from jax.experimental.pallas import tpu_sc as plsc

B = 16384
D = 768
_BM = 1024
_CHUNKS = (2048, 4096, 5120, 5120)
assert sum(_CHUNKS) == B


def _sc_gather(stu_ids, exer_in, stu_table, bert_table, bc):
    info = plsc.get_sparse_core_info()
    nw = info.num_cores * info.num_subcores
    nc = info.num_cores
    bpw = bc // nw
    ch = 64 if bpw % 64 == 0 else 32
    nch = bpw // ch

    mesh = plsc.VectorSubcoreMesh(core_axis_name="c", subcore_axis_name="s")

    @functools.partial(
        pl.kernel,
        mesh=mesh,
        out_type=[
            jax.ShapeDtypeStruct((bc, D), jnp.float32),
            jax.ShapeDtypeStruct((bc, D), jnp.float32),
        ],
        scratch_types=[
            pltpu.VMEM((bpw,), jnp.int32),
            pltpu.VMEM((bpw,), jnp.int32),
            pltpu.VMEM((ch, D), jnp.float32),
            pltpu.VMEM((ch, D), jnp.float32),
            pltpu.SemaphoreType.DMA,
            pltpu.SemaphoreType.DMA,
            pltpu.SemaphoreType.DMA,
            pltpu.SemaphoreType.DMA,
        ],
    )
    def gather_k(stu_ids_hbm, exer_in_hbm, stu_table_hbm, bert_table_hbm,
                 prof_out, exer_out, idx_s, idx_e, rows_0, rows_1,
                 gsem_0, gsem_1, wsem_0, wsem_1):
        wid = lax.axis_index("s") * nc + lax.axis_index("c")
        base = wid * bpw
        pltpu.sync_copy(stu_ids_hbm.at[pl.ds(base, bpw)], idx_s)
        pltpu.sync_copy(exer_in_hbm.at[pl.ds(base, bpw)], idx_e)

        tasks = []
        for c in range(nch):
            tasks.append((idx_s, stu_table_hbm, prof_out, c * ch))
            tasks.append((idx_e, bert_table_hbm, exer_out, c * ch))

        bufs = (rows_0, rows_1)
        gsems = (gsem_0, gsem_1)
        wsems = (wsem_0, wsem_1)

        def issue_gather(t, slot):
            idx, table, _, off = tasks[t]
            return pltpu.async_copy(
                table.at[idx.at[pl.ds(off, ch)]], bufs[slot], gsems[slot])

        gcp = [None, None]
        wcp = [None, None]
        gcp[0] = issue_gather(0, 0)
        for t in range(len(tasks)):
            slot = t % 2
            nslot = (t + 1) % 2
            if t + 1 < len(tasks):
                if wcp[nslot] is not None:
                    wcp[nslot].wait()
                gcp[nslot] = issue_gather(t + 1, nslot)
            gcp[slot].wait()
            _, _, out, off = tasks[t]
            wcp[slot] = pltpu.async_copy(
                bufs[slot], out.at[pl.ds(base + off, ch)], wsems[slot])
        if wcp[0] is not None:
            wcp[0].wait()
        if wcp[1] is not None:
            wcp[1].wait()

    return gather_k(stu_ids, exer_in, stu_table, bert_table)


def _sigmoid(x):
    return 0.5 * jnp.tanh(0.5 * x) + 0.5


def _mlp_impl(ex_ref, pr_ref, w1_ref, b1_ref, w2_ref, b2_ref,
              w3_ref, b3_ref, out_ref, exf_ref, prf_ref):
    ex = ex_ref[...]
    pr = pr_ref[...]
    exf_ref[...] = ex
    prf_ref[...] = pr
    x = jnp.concatenate(
        [ex.astype(jnp.bfloat16), pr.astype(jnp.bfloat16)], axis=1)
    h = jnp.dot(x, w1_ref[...], preferred_element_type=jnp.float32)
    h = _sigmoid(h + b1_ref[...]).astype(jnp.bfloat16)
    h = _sigmoid(
        jnp.dot(h, w2_ref[...], preferred_element_type=jnp.float32)
        + b2_ref[...]).astype(jnp.bfloat16)
    logits = (jnp.dot(h, w3_ref[...], preferred_element_type=jnp.float32)
              + b3_ref[...])
    out_ref[...] = _sigmoid(logits)


def _mlp_body_aliased(ex_ref, pr_ref, w1_ref, b1_ref, w2_ref,
                      b2_ref, w3_ref, b3_ref, of_in, ef_in, pf_in,
                      out_ref, exf_ref, prf_ref):
    del of_in, ef_in, pf_in
    _mlp_impl(ex_ref, pr_ref, w1_ref, b1_ref, w2_ref, b2_ref,
              w3_ref, b3_ref, out_ref, exf_ref, prf_ref)


_WEIGHT_SPECS = [
    pl.BlockSpec((2 * D, 2 * D), lambda i: (0, 0)),
    pl.BlockSpec((1, 2 * D), lambda i: (0, 0)),
    pl.BlockSpec((2 * D, D), lambda i: (0, 0)),
    pl.BlockSpec((1, D), lambda i: (0, 0)),
    pl.BlockSpec((D, 1), lambda i: (0, 0)),
    pl.BlockSpec((1, 1), lambda i: (0, 0)),
]

_OUT_SHAPES = [
    jax.ShapeDtypeStruct((B, 1), jnp.float32),
    jax.ShapeDtypeStruct((B, D), jnp.float32),
    jax.ShapeDtypeStruct((B, D), jnp.float32),
]


def _tc_mlp_chunk(base, bc, ex_k, pr_k, w1, b1r, w2, b2r, w3, b3r,
                  out_full=None, exer_full=None, prof_full=None):
    nsteps = bc // _BM
    boff = base // _BM

    data_specs = [
        pl.BlockSpec((_BM, D), lambda i: (i, 0)),
        pl.BlockSpec((_BM, D), lambda i: (i, 0)),
    ]
    out_specs = [
        pl.BlockSpec((_BM, 1), lambda i: (boff + i, 0)),
        pl.BlockSpec((_BM, D), lambda i: (boff + i, 0)),
        pl.BlockSpec((_BM, D), lambda i: (boff + i, 0)),
    ]
    args = (ex_k, pr_k, w1, b1r, w2, b2r, w3, b3r)
    if out_full is None:
        return pl.pallas_call(
            _mlp_impl,
            grid=(nsteps,),
            in_specs=data_specs + _WEIGHT_SPECS,
            out_specs=out_specs,
            out_shape=_OUT_SHAPES,
        )(*args)
    _full = pl.BlockSpec(memory_space=pl.ANY)
    return pl.pallas_call(
        _mlp_body_aliased,
        grid=(nsteps,),
        in_specs=data_specs + _WEIGHT_SPECS + [_full, _full, _full],
        out_specs=out_specs,
        out_shape=_OUT_SHAPES,
        input_output_aliases={8: 0, 9: 1, 10: 2},
    )(*args, out_full, exer_full, prof_full)


def kernel(stu_ids, exer_in, bert_table, stu_table, W1, b1, W2, b2, W3, b3):
    w1 = W1.astype(jnp.bfloat16)
    w2 = W2.astype(jnp.bfloat16)
    w3 = W3.astype(jnp.bfloat16)
    b1r = b1.reshape(1, 2 * D)
    b2r = b2.reshape(1, D)
    b3r = b3.reshape(1, 1)

    offsets = []
    off = 0
    for bc in _CHUNKS:
        offsets.append(off)
        off += bc

    chunks = []
    for bc, off in zip(_CHUNKS, offsets):
        sl = slice(off, off + bc)
        chunks.append(_sc_gather(stu_ids[sl], exer_in[sl],
                                 stu_table, bert_table, bc))

    out_full = exer_full = prof_full = None
    for (bc, off, (prof_k, exer_k)) in zip(_CHUNKS, offsets, chunks):
        out_full, exer_full, prof_full = _tc_mlp_chunk(
            off, bc, exer_k, prof_k, w1, b1r, w2, b2r, w3, b3r,
            out_full=out_full, exer_full=exer_full, prof_full=prof_full)

    return out_full.reshape(B), exer_full, prof_full


# speedup vs baseline: 1.0923x; 1.0248x over previous
import functools

import jax
import jax.numpy as jnp
from jax import lax
from jax.experimental import pallas as pl
from jax.experimental.pallas import tpu as pltpu
from jax.experimental.pallas import tpu_sc as plsc

B = 16384
D = 768
_BM = 1024
_CHUNKS = (2048, 4096, 5120, 5120)
assert sum(_CHUNKS) == B


def _sc_gather(stu_ids, exer_in, stu_table, bert_table, bc, off):
    info = plsc.get_sparse_core_info()
    nw = info.num_cores * info.num_subcores
    nc = info.num_cores
    bpw = bc // nw
    ch = 32
    nch = bpw // ch
    ntask = 2 * nch
    nbuf = min(4, ntask)

    mesh = plsc.VectorSubcoreMesh(core_axis_name="c", subcore_axis_name="s")

    @functools.partial(
        pl.kernel,
        mesh=mesh,
        out_type=[
            jax.ShapeDtypeStruct((bc, D), jnp.float32),
            jax.ShapeDtypeStruct((bc, D), jnp.float32),
        ],
        scratch_types=(
            [pltpu.VMEM((bpw,), jnp.int32)] * 2
            + [pltpu.VMEM((ch, D), jnp.float32)] * nbuf
            + [pltpu.SemaphoreType.DMA] * (2 * nbuf)
        ),
    )
    def gather_k(stu_ids_hbm, exer_in_hbm, stu_table_hbm, bert_table_hbm,
                 prof_out, exer_out, idx_s, idx_e, *rest):
        bufs = rest[:nbuf]
        gsems = rest[nbuf:2 * nbuf]
        wsems = rest[2 * nbuf:]
        wid = lax.axis_index("s") * nc + lax.axis_index("c")
        base = wid * bpw
        pltpu.sync_copy(stu_ids_hbm.at[pl.ds(off + base, bpw)], idx_s)
        pltpu.sync_copy(exer_in_hbm.at[pl.ds(off + base, bpw)], idx_e)

        tasks = []
        for c in range(nch):
            tasks.append((idx_s, stu_table_hbm, prof_out, c * ch))
            tasks.append((idx_e, bert_table_hbm, exer_out, c * ch))

        def issue_gather(t, slot):
            idx, table, _, o = tasks[t]
            return pltpu.async_copy(
                table.at[idx.at[pl.ds(o, ch)]], bufs[slot], gsems[slot])

        gcp = [None] * nbuf
        wcp = [None] * nbuf
        for t in range(nbuf):
            gcp[t] = issue_gather(t, t)
        for t in range(ntask):
            slot = t % nbuf
            gcp[slot].wait()
            _, _, out, o = tasks[t]
            wcp[slot] = pltpu.async_copy(
                bufs[slot], out.at[pl.ds(base + o, ch)], wsems[slot])
            nt = t + nbuf
            if nt < ntask:
                wcp[slot].wait()
                gcp[slot] = issue_gather(nt, slot)
        for t in range(max(0, ntask - nbuf), ntask):
            wcp[t % nbuf].wait()

    return gather_k(stu_ids, exer_in, stu_table, bert_table)


def _cast_body(w1_ref, w2_ref, w3_ref, o1_ref, o2_ref, o3_ref):
    o1_ref[...] = w1_ref[...].astype(jnp.bfloat16)
    o2_ref[...] = w2_ref[...].astype(jnp.bfloat16)
    o3_ref[...] = w3_ref[...].astype(jnp.bfloat16)


def _prep_weights(W1, W2, W3):
    return pl.pallas_call(
        _cast_body,
        out_shape=[
            jax.ShapeDtypeStruct((2 * D, 2 * D), jnp.bfloat16),
            jax.ShapeDtypeStruct((2 * D, D), jnp.bfloat16),
            jax.ShapeDtypeStruct((D, 1), jnp.bfloat16),
        ],
    )(W1, W2, W3)


def _sigmoid(x):
    return 0.5 * jnp.tanh(0.5 * x) + 0.5


def _mlp_impl(ex_ref, pr_ref, w1_ref, b1_ref, w2_ref, b2_ref,
              w3_ref, b3_ref, out_ref, exf_ref, prf_ref):
    ex = ex_ref[...]
    pr = pr_ref[...]
    exf_ref[...] = ex
    prf_ref[...] = pr
    x = jnp.concatenate(
        [ex.astype(jnp.bfloat16), pr.astype(jnp.bfloat16)], axis=1)
    h = jnp.dot(x, w1_ref[...], preferred_element_type=jnp.float32)
    h = _sigmoid(h + b1_ref[...]).astype(jnp.bfloat16)
    h = _sigmoid(
        jnp.dot(h, w2_ref[...], preferred_element_type=jnp.float32)
        + b2_ref[...]).astype(jnp.bfloat16)
    logits = (jnp.dot(h, w3_ref[...], preferred_element_type=jnp.float32)
              + b3_ref[...])
    out_ref[...] = _sigmoid(logits)


def _mlp_body_aliased(ex_ref, pr_ref, w1_ref, b1_ref, w2_ref,
                      b2_ref, w3_ref, b3_ref, of_in, ef_in, pf_in,
                      out_ref, exf_ref, prf_ref):
    del of_in, ef_in, pf_in
    _mlp_impl(ex_ref, pr_ref, w1_ref, b1_ref, w2_ref, b2_ref,
              w3_ref, b3_ref, out_ref, exf_ref, prf_ref)


_WEIGHT_SPECS = [
    pl.BlockSpec((2 * D, 2 * D), lambda i: (0, 0)),
    pl.BlockSpec((2 * D,), lambda i: (0,)),
    pl.BlockSpec((2 * D, D), lambda i: (0, 0)),
    pl.BlockSpec((D,), lambda i: (0,)),
    pl.BlockSpec((D, 1), lambda i: (0, 0)),
    pl.BlockSpec((1,), lambda i: (0,)),
]

_OUT_SHAPES = [
    jax.ShapeDtypeStruct((B, 1), jnp.float32),
    jax.ShapeDtypeStruct((B, D), jnp.float32),
    jax.ShapeDtypeStruct((B, D), jnp.float32),
]


def _tc_mlp_chunk(base, bc, ex_k, pr_k, w1, b1, w2, b2, w3, b3,
                  out_full=None, exer_full=None, prof_full=None):
    nsteps = bc // _BM
    boff = base // _BM

    data_specs = [
        pl.BlockSpec((_BM, D), lambda i: (i, 0)),
        pl.BlockSpec((_BM, D), lambda i: (i, 0)),
    ]
    out_specs = [
        pl.BlockSpec((_BM, 1), lambda i: (boff + i, 0)),
        pl.BlockSpec((_BM, D), lambda i: (boff + i, 0)),
        pl.BlockSpec((_BM, D), lambda i: (boff + i, 0)),
    ]
    args = (ex_k, pr_k, w1, b1, w2, b2, w3, b3)
    if out_full is None:
        return pl.pallas_call(
            _mlp_impl,
            grid=(nsteps,),
            in_specs=data_specs + _WEIGHT_SPECS,
            out_specs=out_specs,
            out_shape=_OUT_SHAPES,
        )(*args)
    _full = pl.BlockSpec(memory_space=pl.ANY)
    return pl.pallas_call(
        _mlp_body_aliased,
        grid=(nsteps,),
        in_specs=data_specs + _WEIGHT_SPECS + [_full, _full, _full],
        out_specs=out_specs,
        out_shape=_OUT_SHAPES,
        input_output_aliases={8: 0, 9: 1, 10: 2},
    )(*args, out_full, exer_full, prof_full)


def kernel(stu_ids, exer_in, bert_table, stu_table, W1, b1, W2, b2, W3, b3):
    w1, w2, w3 = _prep_weights(W1, W2, W3)

    offsets = []
    off = 0
    for bc in _CHUNKS:
        offsets.append(off)
        off += bc

    chunks = []
    for bc, off in zip(_CHUNKS, offsets):
        chunks.append(_sc_gather(stu_ids, exer_in,
                                 stu_table, bert_table, bc, off))

    out_full = exer_full = prof_full = None
    for (bc, off, (prof_k, exer_k)) in zip(_CHUNKS, offsets, chunks):
        out_full, exer_full, prof_full = _tc_mlp_chunk(
            off, bc, exer_k, prof_k, w1, b1, w2, b2, w3, b3,
            out_full=out_full, exer_full=exer_full, prof_full=prof_full)

    return out_full.reshape(B), exer_full, prof_full


# speedup vs baseline: 1.1019x; 1.0087x over previous
import functools

import jax
import jax.numpy as jnp
from jax import lax
from jax.experimental import pallas as pl
from jax.experimental.pallas import tpu as pltpu
from jax.experimental.pallas import tpu_sc as plsc

B = 16384
D = 768
_BM = 1024
_CHUNKS = (2048, 4096, 5120, 5120)
assert sum(_CHUNKS) == B


def _sc_gather(stu_ids, exer_in, stu_table, bert_table, bc, off):
    info = plsc.get_sparse_core_info()
    nw = info.num_cores * info.num_subcores
    nc = info.num_cores
    bpw = bc // nw
    nch = -(-bpw // 80)
    while bpw % nch:
        nch += 1
    ch = bpw // nch
    ntask = 2 * nch
    nbuf = min(2, ntask)

    mesh = plsc.VectorSubcoreMesh(core_axis_name="c", subcore_axis_name="s")

    @functools.partial(
        pl.kernel,
        mesh=mesh,
        out_type=[
            jax.ShapeDtypeStruct((bc, D), jnp.float32),
            jax.ShapeDtypeStruct((bc, D), jnp.float32),
        ],
        scratch_types=(
            [pltpu.VMEM((bpw,), jnp.int32)] * 2
            + [pltpu.VMEM((ch, D), jnp.float32)] * nbuf
            + [pltpu.SemaphoreType.DMA] * (2 * nbuf)
        ),
    )
    def gather_k(stu_ids_hbm, exer_in_hbm, stu_table_hbm, bert_table_hbm,
                 prof_out, exer_out, idx_s, idx_e, *rest):
        bufs = rest[:nbuf]
        gsems = rest[nbuf:2 * nbuf]
        wsems = rest[2 * nbuf:]
        wid = lax.axis_index("s") * nc + lax.axis_index("c")
        base = wid * bpw
        pltpu.sync_copy(stu_ids_hbm.at[pl.ds(off + base, bpw)], idx_s)
        pltpu.sync_copy(exer_in_hbm.at[pl.ds(off + base, bpw)], idx_e)

        tasks = []
        for c in range(nch):
            tasks.append((idx_s, stu_table_hbm, prof_out, c * ch))
            tasks.append((idx_e, bert_table_hbm, exer_out, c * ch))

        def issue_gather(t, slot):
            idx, table, _, o = tasks[t]
            return pltpu.async_copy(
                table.at[idx.at[pl.ds(o, ch)]], bufs[slot], gsems[slot])

        gcp = [None] * nbuf
        wcp = [None] * nbuf
        for t in range(nbuf):
            gcp[t] = issue_gather(t, t)
        for t in range(ntask):
            slot = t % nbuf
            gcp[slot].wait()
            _, _, out, o = tasks[t]
            wcp[slot] = pltpu.async_copy(
                bufs[slot], out.at[pl.ds(base + o, ch)], wsems[slot])
            nt = t + nbuf
            if nt < ntask:
                wcp[slot].wait()
                gcp[slot] = issue_gather(nt, slot)
        for t in range(max(0, ntask - nbuf), ntask):
            wcp[t % nbuf].wait()

    return gather_k(stu_ids, exer_in, stu_table, bert_table)


def _cast_body(w1_ref, w2_ref, w3_ref, o1_ref, o2_ref, o3_ref):
    o1_ref[...] = w1_ref[...].astype(jnp.bfloat16)
    o2_ref[...] = w2_ref[...].astype(jnp.bfloat16)
    o3_ref[...] = w3_ref[...].astype(jnp.bfloat16)


def _prep_weights(W1, W2, W3):
    g = 4
    r1 = 2 * D // g
    return pl.pallas_call(
        _cast_body,
        grid=(g,),
        in_specs=[
            pl.BlockSpec((r1, 2 * D), lambda i: (i, 0)),
            pl.BlockSpec((r1, D), lambda i: (i, 0)),
            pl.BlockSpec((D // g, 1), lambda i: (i, 0)),
        ],
        out_specs=[
            pl.BlockSpec((r1, 2 * D), lambda i: (i, 0)),
            pl.BlockSpec((r1, D), lambda i: (i, 0)),
            pl.BlockSpec((D // g, 1), lambda i: (i, 0)),
        ],
        out_shape=[
            jax.ShapeDtypeStruct((2 * D, 2 * D), jnp.bfloat16),
            jax.ShapeDtypeStruct((2 * D, D), jnp.bfloat16),
            jax.ShapeDtypeStruct((D, 1), jnp.bfloat16),
        ],
    )(W1, W2, W3)


def _sigmoid(x):
    return 0.5 * jnp.tanh(0.5 * x) + 0.5


def _mlp_impl(ex_ref, pr_ref, w1_ref, b1_ref, w2_ref, b2_ref,
              w3_ref, b3_ref, out_ref, exf_ref, prf_ref):
    ex = ex_ref[...]
    pr = pr_ref[...]
    exf_ref[...] = ex
    prf_ref[...] = pr
    x = jnp.concatenate(
        [ex.astype(jnp.bfloat16), pr.astype(jnp.bfloat16)], axis=1)
    h = jnp.dot(x, w1_ref[...], preferred_element_type=jnp.float32)
    h = _sigmoid(h + b1_ref[...]).astype(jnp.bfloat16)
    h = _sigmoid(
        jnp.dot(h, w2_ref[...], preferred_element_type=jnp.float32)
        + b2_ref[...]).astype(jnp.bfloat16)
    logits = (jnp.dot(h, w3_ref[...], preferred_element_type=jnp.float32)
              + b3_ref[...])
    out_ref[...] = _sigmoid(logits)


def _mlp_body_aliased(ex_ref, pr_ref, w1_ref, b1_ref, w2_ref,
                      b2_ref, w3_ref, b3_ref, of_in, ef_in, pf_in,
                      out_ref, exf_ref, prf_ref):
    del of_in, ef_in, pf_in
    _mlp_impl(ex_ref, pr_ref, w1_ref, b1_ref, w2_ref, b2_ref,
              w3_ref, b3_ref, out_ref, exf_ref, prf_ref)


_WEIGHT_SPECS = [
    pl.BlockSpec((2 * D, 2 * D), lambda i: (0, 0)),
    pl.BlockSpec((2 * D,), lambda i: (0,)),
    pl.BlockSpec((2 * D, D), lambda i: (0, 0)),
    pl.BlockSpec((D,), lambda i: (0,)),
    pl.BlockSpec((D, 1), lambda i: (0, 0)),
    pl.BlockSpec((1,), lambda i: (0,)),
]

_OUT_SHAPES = [
    jax.ShapeDtypeStruct((B, 1), jnp.float32),
    jax.ShapeDtypeStruct((B, D), jnp.float32),
    jax.ShapeDtypeStruct((B, D), jnp.float32),
]


def _tc_mlp_chunk(base, bc, ex_k, pr_k, w1, b1, w2, b2, w3, b3,
                  out_full=None, exer_full=None, prof_full=None):
    nsteps = bc // _BM
    boff = base // _BM

    data_specs = [
        pl.BlockSpec((_BM, D), lambda i: (i, 0)),
        pl.BlockSpec((_BM, D), lambda i: (i, 0)),
    ]
    out_specs = [
        pl.BlockSpec((_BM, 1), lambda i: (boff + i, 0)),
        pl.BlockSpec((_BM, D), lambda i: (boff + i, 0)),
        pl.BlockSpec((_BM, D), lambda i: (boff + i, 0)),
    ]
    args = (ex_k, pr_k, w1, b1, w2, b2, w3, b3)
    if out_full is None:
        return pl.pallas_call(
            _mlp_impl,
            grid=(nsteps,),
            in_specs=data_specs + _WEIGHT_SPECS,
            out_specs=out_specs,
            out_shape=_OUT_SHAPES,
        )(*args)
    _full = pl.BlockSpec(memory_space=pl.ANY)
    return pl.pallas_call(
        _mlp_body_aliased,
        grid=(nsteps,),
        in_specs=data_specs + _WEIGHT_SPECS + [_full, _full, _full],
        out_specs=out_specs,
        out_shape=_OUT_SHAPES,
        input_output_aliases={8: 0, 9: 1, 10: 2},
    )(*args, out_full, exer_full, prof_full)


def kernel(stu_ids, exer_in, bert_table, stu_table, W1, b1, W2, b2, W3, b3):
    w1, w2, w3 = _prep_weights(W1, W2, W3)

    offsets = []
    off = 0
    for bc in _CHUNKS:
        offsets.append(off)
        off += bc

    chunks = []
    for bc, off in zip(_CHUNKS, offsets):
        chunks.append(_sc_gather(stu_ids, exer_in,
                                 stu_table, bert_table, bc, off))

    out_full = exer_full = prof_full = None
    for (bc, off, (prof_k, exer_k)) in zip(_CHUNKS, offsets, chunks):
        out_full, exer_full, prof_full = _tc_mlp_chunk(
            off, bc, exer_k, prof_k, w1, b1, w2, b2, w3, b3,
            out_full=out_full, exer_full=exer_full, prof_full=prof_full)

    return out_full.reshape(B), exer_full, prof_full
